# baseline (speedup 1.0000x reference)
import jax
import jax.numpy as jnp
from jax.experimental import pallas as pl
from jax.experimental.pallas import tpu as pltpu

_SEQ_BLOCK = 512


def _make_body(batch, n_blocks):
    assert n_blocks % 2 == 0
    half = n_blocks // 2

    def body(pe_hbm, out_hbm, buf0, buf1, rsem0, rsem1, wsem0, wsem1):
        j = pl.program_id(0)
        bufs, rsems, wsems = (buf0, buf1), (rsem0, rsem1), (wsem0, wsem1)

        def read_copy(step, parity):
            return pltpu.make_async_copy(
                pe_hbm.at[pl.ds(step * _SEQ_BLOCK, _SEQ_BLOCK)],
                bufs[parity], rsems[parity])

        def write_copies(step, parity):
            return [
                pltpu.make_async_copy(
                    bufs[parity],
                    out_hbm.at[b, pl.ds(step * _SEQ_BLOCK, _SEQ_BLOCK)],
                    wsems[parity])
                for b in range(batch)
            ]

        @pl.when(j == 0)
        def _():
            read_copy(0, 0).start()

        @pl.when(j > 0)
        def _():
            for c in write_copies(2 * j - 1, 1):
                c.wait()

        read_copy(2 * j + 1, 1).start()
        read_copy(2 * j, 0).wait()
        for c in write_copies(2 * j, 0):
            c.start()

        @pl.when(j < half - 1)
        def _():
            for c in write_copies(2 * j, 0):
                c.wait()
            read_copy(2 * j + 2, 0).start()

        read_copy(2 * j + 1, 1).wait()
        for c in write_copies(2 * j + 1, 1):
            c.start()

        @pl.when(j == half - 1)
        def _():
            for c in write_copies(2 * j, 0):
                c.wait()
            for c in write_copies(2 * j + 1, 1):
                c.wait()
    return body


def kernel(inputs, position_embeddings):
    batch, seq, width = inputs.shape
    pe = position_embeddings[:seq, :]
    n_blocks = seq // _SEQ_BLOCK
    out = pl.pallas_call(
        _make_body(batch, n_blocks),
        grid=(n_blocks // 2,),
        in_specs=[pl.BlockSpec(memory_space=pl.ANY)],
        out_specs=pl.BlockSpec(memory_space=pl.ANY),
        out_shape=jax.ShapeDtypeStruct((batch, seq, width), jnp.float32),
        scratch_shapes=[
            pltpu.VMEM((_SEQ_BLOCK, width), jnp.float32),
            pltpu.VMEM((_SEQ_BLOCK, width), jnp.float32),
            pltpu.SemaphoreType.DMA,
            pltpu.SemaphoreType.DMA,
            pltpu.SemaphoreType.DMA,
            pltpu.SemaphoreType.DMA,
        ],
    )(pe)
    return out


# speedup vs baseline: 1.1087x; 1.1087x over previous
import jax
import jax.numpy as jnp
from jax.experimental import pallas as pl
from jax.experimental.pallas import tpu as pltpu

_SEQ_BLOCK = 1024


def _make_body(batch, n_blocks):
    assert n_blocks % 2 == 0
    half = n_blocks // 2

    def body(pe_hbm, out_hbm, buf0, buf1, rsem0, rsem1, wsem0, wsem1):
        j = pl.program_id(0)
        bufs, rsems, wsems = (buf0, buf1), (rsem0, rsem1), (wsem0, wsem1)

        def read_copy(step, parity):
            return pltpu.make_async_copy(
                pe_hbm.at[pl.ds(step * _SEQ_BLOCK, _SEQ_BLOCK)],
                bufs[parity], rsems[parity])

        def write_copies(step, parity):
            return [
                pltpu.make_async_copy(
                    bufs[parity],
                    out_hbm.at[b, pl.ds(step * _SEQ_BLOCK, _SEQ_BLOCK)],
                    wsems[parity])
                for b in range(batch)
            ]

        @pl.when(j == 0)
        def _():
            read_copy(0, 0).start()

        @pl.when(j > 0)
        def _():
            for c in write_copies(2 * j - 1, 1):
                c.wait()

        read_copy(2 * j + 1, 1).start()
        read_copy(2 * j, 0).wait()
        for c in write_copies(2 * j, 0):
            c.start()

        @pl.when(j < half - 1)
        def _():
            for c in write_copies(2 * j, 0):
                c.wait()
            read_copy(2 * j + 2, 0).start()

        read_copy(2 * j + 1, 1).wait()
        for c in write_copies(2 * j + 1, 1):
            c.start()

        @pl.when(j == half - 1)
        def _():
            for c in write_copies(2 * j, 0):
                c.wait()
            for c in write_copies(2 * j + 1, 1):
                c.wait()
    return body


def kernel(inputs, position_embeddings):
    batch, seq, width = inputs.shape
    pe = position_embeddings[:seq, :]
    n_blocks = seq // _SEQ_BLOCK
    out = pl.pallas_call(
        _make_body(batch, n_blocks),
        grid=(n_blocks // 2,),
        in_specs=[pl.BlockSpec(memory_space=pl.ANY)],
        out_specs=pl.BlockSpec(memory_space=pl.ANY),
        out_shape=jax.ShapeDtypeStruct((batch, seq, width), jnp.float32),
        scratch_shapes=[
            pltpu.VMEM((_SEQ_BLOCK, width), jnp.float32),
            pltpu.VMEM((_SEQ_BLOCK, width), jnp.float32),
            pltpu.SemaphoreType.DMA,
            pltpu.SemaphoreType.DMA,
            pltpu.SemaphoreType.DMA,
            pltpu.SemaphoreType.DMA,
        ],
    )(pe)
    return out


# speedup vs baseline: 1.1737x; 1.0587x over previous
import jax
import jax.numpy as jnp
from jax.experimental import pallas as pl
from jax.experimental.pallas import tpu as pltpu

_SEQ_BLOCK = 2048


def _make_body(batch, n_blocks):
    assert n_blocks % 2 == 0
    half = n_blocks // 2

    def body(pe_hbm, out_hbm, buf0, buf1, rsem0, rsem1, wsem0, wsem1):
        j = pl.program_id(0)
        bufs, rsems, wsems = (buf0, buf1), (rsem0, rsem1), (wsem0, wsem1)

        def read_copy(step, parity):
            return pltpu.make_async_copy(
                pe_hbm.at[pl.ds(step * _SEQ_BLOCK, _SEQ_BLOCK)],
                bufs[parity], rsems[parity])

        def write_copies(step, parity):
            return [
                pltpu.make_async_copy(
                    bufs[parity],
                    out_hbm.at[b, pl.ds(step * _SEQ_BLOCK, _SEQ_BLOCK)],
                    wsems[parity])
                for b in range(batch)
            ]

        @pl.when(j == 0)
        def _():
            read_copy(0, 0).start()

        @pl.when(j > 0)
        def _():
            for c in write_copies(2 * j - 1, 1):
                c.wait()

        read_copy(2 * j + 1, 1).start()
        read_copy(2 * j, 0).wait()
        for c in write_copies(2 * j, 0):
            c.start()

        @pl.when(j < half - 1)
        def _():
            for c in write_copies(2 * j, 0):
                c.wait()
            read_copy(2 * j + 2, 0).start()

        read_copy(2 * j + 1, 1).wait()
        for c in write_copies(2 * j + 1, 1):
            c.start()

        @pl.when(j == half - 1)
        def _():
            for c in write_copies(2 * j, 0):
                c.wait()
            for c in write_copies(2 * j + 1, 1):
                c.wait()
    return body


def kernel(inputs, position_embeddings):
    batch, seq, width = inputs.shape
    pe = position_embeddings[:seq, :]
    n_blocks = seq // _SEQ_BLOCK
    out = pl.pallas_call(
        _make_body(batch, n_blocks),
        grid=(n_blocks // 2,),
        in_specs=[pl.BlockSpec(memory_space=pl.ANY)],
        out_specs=pl.BlockSpec(memory_space=pl.ANY),
        out_shape=jax.ShapeDtypeStruct((batch, seq, width), jnp.float32),
        scratch_shapes=[
            pltpu.VMEM((_SEQ_BLOCK, width), jnp.float32),
            pltpu.VMEM((_SEQ_BLOCK, width), jnp.float32),
            pltpu.SemaphoreType.DMA,
            pltpu.SemaphoreType.DMA,
            pltpu.SemaphoreType.DMA,
            pltpu.SemaphoreType.DMA,
        ],
    )(pe)
    return out


# speedup vs baseline: 1.2087x; 1.0298x over previous
import jax
import jax.numpy as jnp
from jax.experimental import pallas as pl
from jax.experimental.pallas import tpu as pltpu

_SEQ_BLOCK = 4096


def _make_body(batch, n_blocks):
    assert n_blocks % 2 == 0
    half = n_blocks // 2

    def body(pe_hbm, out_hbm, buf0, buf1, rsem0, rsem1, wsem0, wsem1):
        j = pl.program_id(0)
        bufs, rsems, wsems = (buf0, buf1), (rsem0, rsem1), (wsem0, wsem1)

        def read_copy(step, parity):
            return pltpu.make_async_copy(
                pe_hbm.at[pl.ds(step * _SEQ_BLOCK, _SEQ_BLOCK)],
                bufs[parity], rsems[parity])

        def write_copies(step, parity):
            return [
                pltpu.make_async_copy(
                    bufs[parity],
                    out_hbm.at[b, pl.ds(step * _SEQ_BLOCK, _SEQ_BLOCK)],
                    wsems[parity])
                for b in range(batch)
            ]

        @pl.when(j == 0)
        def _():
            read_copy(0, 0).start()

        @pl.when(j > 0)
        def _():
            for c in write_copies(2 * j - 1, 1):
                c.wait()

        read_copy(2 * j + 1, 1).start()
        read_copy(2 * j, 0).wait()
        for c in write_copies(2 * j, 0):
            c.start()

        @pl.when(j < half - 1)
        def _():
            for c in write_copies(2 * j, 0):
                c.wait()
            read_copy(2 * j + 2, 0).start()

        read_copy(2 * j + 1, 1).wait()
        for c in write_copies(2 * j + 1, 1):
            c.start()

        @pl.when(j == half - 1)
        def _():
            for c in write_copies(2 * j, 0):
                c.wait()
            for c in write_copies(2 * j + 1, 1):
                c.wait()
    return body


def kernel(inputs, position_embeddings):
    batch, seq, width = inputs.shape
    pe = position_embeddings[:seq, :]
    n_blocks = seq // _SEQ_BLOCK
    out = pl.pallas_call(
        _make_body(batch, n_blocks),
        grid=(n_blocks // 2,),
        in_specs=[pl.BlockSpec(memory_space=pl.ANY)],
        out_specs=pl.BlockSpec(memory_space=pl.ANY),
        out_shape=jax.ShapeDtypeStruct((batch, seq, width), jnp.float32),
        scratch_shapes=[
            pltpu.VMEM((_SEQ_BLOCK, width), jnp.float32),
            pltpu.VMEM((_SEQ_BLOCK, width), jnp.float32),
            pltpu.SemaphoreType.DMA,
            pltpu.SemaphoreType.DMA,
            pltpu.SemaphoreType.DMA,
            pltpu.SemaphoreType.DMA,
        ],
    )(pe)
    return out
